# Initial kernel scaffold; baseline (speedup 1.0000x reference)
import jax
import jax.numpy as jnp
from jax.experimental import pallas as pl


def kernel(node_states, edge_index, edge, a_in, bias):
    raise NotImplementedError("write your pallas kernel here")


# speedup vs baseline: 1.8096x; 1.8096x over previous
import functools

import jax
import jax.numpy as jnp
from jax.experimental import pallas as pl
from jax.experimental.pallas import tpu as pltpu

N_NODES = 10000
N_EDGES = 160000
D = 32
DD = D * D


_MV_BLK = 2000


def _matvec_body(x_ref, a_ref, r_ref, out_ref):
    xb = x_ref[...].astype(jnp.bfloat16)
    xrep = jnp.dot(xb, r_ref[...], preferred_element_type=jnp.float32)
    prod = xrep * a_ref[...]
    t = prod[:, 0:128]
    for g in range(1, 8):
        t = t + prod[:, g * 128:(g + 1) * 128]
    out_ref[...] = (t[:, 0:32] + t[:, 32:64] + t[:, 64:96] + t[:, 96:128])


def _matvec(x_i, a2, r_mat):
    grid = N_EDGES // _MV_BLK
    return pl.pallas_call(
        _matvec_body,
        grid=(grid,),
        in_specs=[
            pl.BlockSpec((_MV_BLK, D), lambda i: (i, 0)),
            pl.BlockSpec((_MV_BLK, DD), lambda i: (i, 0)),
            pl.BlockSpec((D, DD), lambda i: (0, 0)),
        ],
        out_specs=pl.BlockSpec((_MV_BLK, D), lambda i: (i, 0)),
        out_shape=jax.ShapeDtypeStruct((N_EDGES, D), jnp.float32),
        compiler_params=pltpu.CompilerParams(
            dimension_semantics=("arbitrary",),
        ),
    )(x_i, a2, r_mat)


def _make_r() -> jax.Array:
    k = jnp.arange(D)[:, None]
    c = jnp.arange(DD)[None, :]
    return (c // D == k).astype(jnp.bfloat16)


_FIN_BLK = 2000


def _finalize_body(s_ref, c_ref, b_ref, out_ref):
    s = s_ref[0] + s_ref[1]
    c = c_ref[0] + c_ref[1]
    mean = s / jnp.maximum(c, 1.0)
    out_ref[...] = jnp.maximum(mean + b_ref[...], 0.0)


def _finalize(sums, counts, bias):
    grid = N_NODES // _FIN_BLK
    return pl.pallas_call(
        _finalize_body,
        grid=(grid,),
        in_specs=[
            pl.BlockSpec((2, _FIN_BLK, D), lambda i: (0, i, 0)),
            pl.BlockSpec((2, _FIN_BLK, 1), lambda i: (0, i, 0)),
            pl.BlockSpec((1, D), lambda i: (0, 0)),
        ],
        out_specs=pl.BlockSpec((_FIN_BLK, D), lambda i: (i, 0)),
        out_shape=jax.ShapeDtypeStruct((N_NODES, D), jnp.float32),
    )(sums, counts, bias.reshape(1, D))


def kernel(node_states, edge_index, edge, a_in, bias):
    del edge
    src = edge_index[:, 0]
    dst = edge_index[:, 1]
    a2 = a_in.reshape(N_EDGES, DD)

    x_i = jnp.take(node_states, src, axis=0)

    msg = _matvec(x_i, a2, _make_r())

    sums = jax.ops.segment_sum(msg, dst, num_segments=N_NODES)
    counts = jax.ops.segment_sum(
        jnp.ones((N_EDGES,), jnp.float32), dst, num_segments=N_NODES)
    sums2 = jnp.stack([sums, jnp.zeros_like(sums)])
    counts2 = jnp.stack([counts, jnp.zeros_like(counts)]).reshape(2, N_NODES, 1)

    return _finalize(sums2, counts2, bias)


# speedup vs baseline: 2.1654x; 1.1966x over previous
import functools

import jax
import jax.numpy as jnp
from jax import lax
from jax.experimental import pallas as pl
from jax.experimental.pallas import tpu as pltpu
from jax.experimental.pallas import tpu_sc as plsc

N_NODES = 10000
N_EDGES = 160000
D = 32
DD = D * D

_NC = 2
_NS = 16
_NW = _NC * _NS
_CHUNK = 128
_NCHUNKS = N_EDGES // _CHUNK
_SC_MESH = dict(core_axis_name="c", subcore_axis_name="s",
                num_cores=_NC, num_subcores=_NS)


def _gather_body(ns_hbm, src_hbm, out_hbm, idx_v, rows_v, sem):
    wid = lax.axis_index("c") * _NS + lax.axis_index("s")

    def step(t, _):
        chunk = wid + t * _NW

        @pl.when(chunk < _NCHUNKS)
        def _():
            pltpu.sync_copy(src_hbm.at[chunk], idx_v)
            pltpu.async_copy(ns_hbm.at[idx_v], rows_v, sem).wait()
            pltpu.sync_copy(rows_v, out_hbm.at[pl.ds(chunk * _CHUNK, _CHUNK)])
        return _

    lax.fori_loop(0, (_NCHUNKS + _NW - 1) // _NW, step, None)


def _sc_gather(node_states, src2):
    mesh = plsc.VectorSubcoreMesh(**_SC_MESH)
    return pl.kernel(
        _gather_body,
        out_type=jax.ShapeDtypeStruct((N_EDGES, D), jnp.float32),
        mesh=mesh,
        scratch_types=[
            pltpu.VMEM((_CHUNK,), jnp.int32),
            pltpu.VMEM((_CHUNK, D), jnp.float32),
            pltpu.SemaphoreType.DMA,
        ],
        compiler_params=pltpu.CompilerParams(use_tc_tiling_on_sc=False),
    )(node_states, src2)


_MV_BLK = 2000


def _matvec_body(x_ref, a_ref, r_ref, out_ref):
    xb = x_ref[...].astype(jnp.bfloat16)
    xrep = jnp.dot(xb, r_ref[...], preferred_element_type=jnp.float32)
    prod = xrep * a_ref[...]
    t = prod[:, 0:128]
    for g in range(1, 8):
        t = t + prod[:, g * 128:(g + 1) * 128]
    out_ref[...] = (t[:, 0:32] + t[:, 32:64] + t[:, 64:96] + t[:, 96:128])


def _matvec(x_i, a2, r_mat):
    grid = N_EDGES // _MV_BLK
    return pl.pallas_call(
        _matvec_body,
        grid=(grid,),
        in_specs=[
            pl.BlockSpec((_MV_BLK, D), lambda i: (i, 0)),
            pl.BlockSpec((_MV_BLK, DD), lambda i: (i, 0)),
            pl.BlockSpec((D, DD), lambda i: (0, 0)),
        ],
        out_specs=pl.BlockSpec((_MV_BLK, D), lambda i: (i, 0)),
        out_shape=jax.ShapeDtypeStruct((N_EDGES, D), jnp.float32),
        compiler_params=pltpu.CompilerParams(
            dimension_semantics=("arbitrary",),
        ),
    )(x_i, a2, r_mat)


def _make_r() -> jax.Array:
    k = jnp.arange(D)[:, None]
    c = jnp.arange(DD)[None, :]
    return (c // D == k).astype(jnp.bfloat16)


_FIN_BLK = 2000


def _finalize_body(s_ref, c_ref, b_ref, out_ref):
    s = s_ref[0] + s_ref[1]
    c = c_ref[0] + c_ref[1]
    mean = s / jnp.maximum(c, 1.0)
    out_ref[...] = jnp.maximum(mean + b_ref[...], 0.0)


def _finalize(sums, counts, bias):
    grid = N_NODES // _FIN_BLK
    return pl.pallas_call(
        _finalize_body,
        grid=(grid,),
        in_specs=[
            pl.BlockSpec((2, _FIN_BLK, D), lambda i: (0, i, 0)),
            pl.BlockSpec((2, _FIN_BLK, 1), lambda i: (0, i, 0)),
            pl.BlockSpec((1, D), lambda i: (0, 0)),
        ],
        out_specs=pl.BlockSpec((_FIN_BLK, D), lambda i: (i, 0)),
        out_shape=jax.ShapeDtypeStruct((N_NODES, D), jnp.float32),
    )(sums, counts, bias.reshape(1, D))


def kernel(node_states, edge_index, edge, a_in, bias):
    del edge
    src = edge_index[:, 0]
    dst = edge_index[:, 1]
    a2 = a_in.reshape(N_EDGES, DD)

    x_i = _sc_gather(node_states, src.reshape(_NCHUNKS, _CHUNK))

    msg = _matvec(x_i, a2, _make_r())

    sums = jax.ops.segment_sum(msg, dst, num_segments=N_NODES)
    counts = jax.ops.segment_sum(
        jnp.ones((N_EDGES,), jnp.float32), dst, num_segments=N_NODES)
    sums2 = jnp.stack([sums, jnp.zeros_like(sums)])
    counts2 = jnp.stack([counts, jnp.zeros_like(counts)]).reshape(2, N_NODES, 1)

    return _finalize(sums2, counts2, bias)


# speedup vs baseline: 3.4155x; 1.5773x over previous
import functools

import jax
import jax.numpy as jnp
from jax import lax
from jax.experimental import pallas as pl
from jax.experimental.pallas import tpu as pltpu
from jax.experimental.pallas import tpu_sc as plsc

N_NODES = 10000
N_EDGES = 160000
D = 32
DD = D * D

_NC = 2
_NS = 16
_NW = _NC * _NS
_CHUNK = 128
_NCHUNKS = N_EDGES // _CHUNK
_SC_MESH = dict(core_axis_name="c", subcore_axis_name="s",
                num_cores=_NC, num_subcores=_NS)


def _gather_body(ns_hbm, src_hbm, out_hbm, idx_v, rows_v, sem):
    wid = lax.axis_index("c") * _NS + lax.axis_index("s")

    def step(t, _):
        chunk = wid + t * _NW

        @pl.when(chunk < _NCHUNKS)
        def _():
            pltpu.sync_copy(src_hbm.at[chunk], idx_v)
            pltpu.async_copy(ns_hbm.at[idx_v], rows_v, sem).wait()
            pltpu.sync_copy(rows_v, out_hbm.at[pl.ds(chunk * _CHUNK, _CHUNK)])
        return _

    lax.fori_loop(0, (_NCHUNKS + _NW - 1) // _NW, step, None)


def _sc_gather(node_states, src2):
    mesh = plsc.VectorSubcoreMesh(**_SC_MESH)
    return pl.kernel(
        _gather_body,
        out_type=jax.ShapeDtypeStruct((N_EDGES, D), jnp.float32),
        mesh=mesh,
        scratch_types=[
            pltpu.VMEM((_CHUNK,), jnp.int32),
            pltpu.VMEM((_CHUNK, D), jnp.float32),
            pltpu.SemaphoreType.DMA,
        ],
        compiler_params=pltpu.CompilerParams(use_tc_tiling_on_sc=False),
    )(node_states, src2)


_ROWS_PER_SUB = N_NODES // _NS
_CNT_STEP = 624
_CNT_WIN = 640


def _scatter_body(msg_hbm, dst_hbm, zs_hbm, zc_hbm, sums_hbm, cnts_hbm,
                  acc_sh, cnt_sh, idx_v, msg_v, ones_v, row_v, col_v):
    cid = lax.axis_index("c")
    sid = lax.axis_index("s")
    wid = cid * _NS + sid
    rows0 = sid * _ROWS_PER_SUB
    c0 = sid * _CNT_STEP

    pltpu.sync_copy(zs_hbm.at[pl.ds(rows0, _ROWS_PER_SUB)], row_v)
    pltpu.sync_copy(row_v, acc_sh.at[pl.ds(rows0, _ROWS_PER_SUB)])
    pltpu.sync_copy(zc_hbm.at[pl.ds(c0, _CNT_WIN)], col_v)
    pltpu.sync_copy(col_v, cnt_sh.at[pl.ds(c0, _CNT_WIN)])
    for i in range(_CHUNK // 16):
        ones_v[pl.ds(i * 16, 16)] = jnp.full((16,), 1.0, jnp.float32)
    plsc.subcore_barrier()

    def step(t, _):
        chunk = wid + t * _NW

        @pl.when(chunk < _NCHUNKS)
        def _():
            pltpu.sync_copy(dst_hbm.at[chunk], idx_v)
            pltpu.sync_copy(msg_hbm.at[pl.ds(chunk * _CHUNK, _CHUNK)], msg_v)
            pltpu.sync_copy(msg_v, acc_sh.at[idx_v], add=True)
            pltpu.sync_copy(ones_v, cnt_sh.at[idx_v], add=True)
        return _

    lax.fori_loop(0, (_NCHUNKS + _NW - 1) // _NW, step, None)
    plsc.subcore_barrier()

    pltpu.sync_copy(acc_sh.at[pl.ds(rows0, _ROWS_PER_SUB)], row_v)
    pltpu.sync_copy(row_v, sums_hbm.at[cid, pl.ds(rows0, _ROWS_PER_SUB)])
    pltpu.sync_copy(cnt_sh.at[pl.ds(c0, _CNT_WIN)], col_v)
    pltpu.sync_copy(col_v, cnts_hbm.at[cid, pl.ds(c0, _CNT_WIN)])


def _sc_scatter(msg, dst2):
    mesh = plsc.VectorSubcoreMesh(**_SC_MESH)
    zs = jnp.zeros((N_NODES, D), jnp.float32)
    zc = jnp.zeros((N_NODES,), jnp.float32)
    return pl.kernel(
        _scatter_body,
        out_type=(
            jax.ShapeDtypeStruct((_NC, N_NODES, D), jnp.float32),
            jax.ShapeDtypeStruct((_NC, N_NODES), jnp.float32),
        ),
        mesh=mesh,
        scratch_types=[
            pltpu.VMEM_SHARED((N_NODES, D), jnp.float32),
            pltpu.VMEM_SHARED((N_NODES,), jnp.float32),
            pltpu.VMEM((_CHUNK,), jnp.int32),
            pltpu.VMEM((_CHUNK, D), jnp.float32),
            pltpu.VMEM((_CHUNK,), jnp.float32),
            pltpu.VMEM((_ROWS_PER_SUB, D), jnp.float32),
            pltpu.VMEM((_CNT_WIN,), jnp.float32),
        ],
        compiler_params=pltpu.CompilerParams(use_tc_tiling_on_sc=False),
    )(msg, dst2, zs, zc)


_MV_BLK = 2000


def _matvec_body(x_ref, a_ref, r_ref, out_ref):
    xb = x_ref[...].astype(jnp.bfloat16)
    xrep = jnp.dot(xb, r_ref[...], preferred_element_type=jnp.float32)
    prod = xrep * a_ref[...]
    t = prod[:, 0:128]
    for g in range(1, 8):
        t = t + prod[:, g * 128:(g + 1) * 128]
    out_ref[...] = (t[:, 0:32] + t[:, 32:64] + t[:, 64:96] + t[:, 96:128])


def _matvec(x_i, a2, r_mat):
    grid = N_EDGES // _MV_BLK
    return pl.pallas_call(
        _matvec_body,
        grid=(grid,),
        in_specs=[
            pl.BlockSpec((_MV_BLK, D), lambda i: (i, 0)),
            pl.BlockSpec((_MV_BLK, DD), lambda i: (i, 0)),
            pl.BlockSpec((D, DD), lambda i: (0, 0)),
        ],
        out_specs=pl.BlockSpec((_MV_BLK, D), lambda i: (i, 0)),
        out_shape=jax.ShapeDtypeStruct((N_EDGES, D), jnp.float32),
        compiler_params=pltpu.CompilerParams(
            dimension_semantics=("arbitrary",),
        ),
    )(x_i, a2, r_mat)


def _make_r() -> jax.Array:
    k = jnp.arange(D)[:, None]
    c = jnp.arange(DD)[None, :]
    return (c // D == k).astype(jnp.bfloat16)


_FIN_BLK = 2000


def _finalize_body(s_ref, c_ref, b_ref, out_ref):
    s = s_ref[0] + s_ref[1]
    c = c_ref[0] + c_ref[1]
    mean = s / jnp.maximum(c, 1.0)
    out_ref[...] = jnp.maximum(mean + b_ref[...], 0.0)


def _finalize(sums, counts, bias):
    grid = N_NODES // _FIN_BLK
    return pl.pallas_call(
        _finalize_body,
        grid=(grid,),
        in_specs=[
            pl.BlockSpec((2, _FIN_BLK, D), lambda i: (0, i, 0)),
            pl.BlockSpec((2, _FIN_BLK, 1), lambda i: (0, i, 0)),
            pl.BlockSpec((1, D), lambda i: (0, 0)),
        ],
        out_specs=pl.BlockSpec((_FIN_BLK, D), lambda i: (i, 0)),
        out_shape=jax.ShapeDtypeStruct((N_NODES, D), jnp.float32),
    )(sums, counts, bias.reshape(1, D))


def kernel(node_states, edge_index, edge, a_in, bias):
    del edge
    src = edge_index[:, 0]
    dst = edge_index[:, 1]
    a2 = a_in.reshape(N_EDGES, DD)

    x_i = _sc_gather(node_states, src.reshape(_NCHUNKS, _CHUNK))

    msg = _matvec(x_i, a2, _make_r())

    sums2, counts2 = _sc_scatter(msg, dst.reshape(_NCHUNKS, _CHUNK))
    return _finalize(sums2, counts2.reshape(_NC, N_NODES, 1), bias)
